# Initial kernel scaffold; baseline (speedup 1.0000x reference)
import jax
import jax.numpy as jnp
from jax import lax
from jax.experimental import pallas as pl
from jax.experimental.pallas import tpu as pltpu
from jax.experimental.pallas import tpu_sc as plsc

B, D, K, M = 16384, 128, 1000, 30
ENT_THRESHOLD = 0.6
KP = 1024
BLK = 512
NB = B // BLK

NC, NS = 2, 16
NW = NC * NS
CPW = KP // NW
BIGI = jnp.int32(2**30)


def _k1_body(z_ref, W_ref, b_ref, ent_ref, yh_ref, zn_ref, tot_ref):
    i = pl.program_id(0)
    z = z_ref[...]
    Wm = W_ref[...]
    logits = lax.dot_general(z, Wm, (((1,), (1,)), ((), ())),
                             preferred_element_type=jnp.float32)
    logits = logits + b_ref[...]
    m = jnp.max(logits, axis=1, keepdims=True)
    ex = jnp.exp(logits - m)
    S = jnp.sum(ex, axis=1, keepdims=True)
    ent = jnp.log(S) - jnp.sum(ex * (logits - m), axis=1, keepdims=True) / S
    cols = lax.broadcasted_iota(jnp.int32, logits.shape, 1)
    yh = jnp.min(jnp.where(logits == m, cols, K), axis=1, keepdims=True)
    nrm = jnp.sqrt(jnp.sum(z * z, axis=1, keepdims=True))
    zn = z / jnp.maximum(nrm, 1e-12)

    ent_ref[...] = ent
    yh_ref[...] = yh
    zn_ref[...] = zn
    kc = jnp.sum((ent <= ENT_THRESHOLD).astype(jnp.int32))

    @pl.when(i == 0)
    def _():
        tot_ref[0, 0] = kc

    @pl.when(i > 0)
    def _():
        tot_ref[0, 0] += kc


_k1 = pl.pallas_call(
    _k1_body,
    grid=(NB,),
    in_specs=[
        pl.BlockSpec((BLK, D), lambda i: (i, 0)),
        pl.BlockSpec((K, D), lambda i: (0, 0)),
        pl.BlockSpec((1, K), lambda i: (0, 0)),
    ],
    out_specs=[
        pl.BlockSpec((BLK, 1), lambda i: (i, 0)),
        pl.BlockSpec((BLK, 1), lambda i: (i, 0)),
        pl.BlockSpec((BLK, D), lambda i: (i, 0)),
        pl.BlockSpec(memory_space=pltpu.SMEM),
    ],
    out_shape=[
        jax.ShapeDtypeStruct((B, 1), jnp.float32),
        jax.ShapeDtypeStruct((B, 1), jnp.int32),
        jax.ShapeDtypeStruct((B, D), jnp.float32),
        jax.ShapeDtypeStruct((1, 1), jnp.int32),
    ],
)


def _sc_body(tot_hbm, ent_hbm, yh_hbm, zn_hbm, acc_hbm, cnt_hbm,
             tot_v, ent_v, yh_v, row_v, accrow_v, acc_v, cnt_v):
    c = lax.axis_index("c")
    s = lax.axis_index("s")
    wid = s * NC + c
    base_k = wid * CPW

    pltpu.sync_copy(tot_hbm, tot_v)
    t = jnp.max(tot_v[...])

    zeros16 = jnp.zeros((16,), jnp.float32)

    def zero_acc(j, _):
        acc_v[pl.ds(j * 16, 16)] = zeros16
        return 0
    lax.fori_loop(0, CPW * D // 16, zero_acc, 0)

    def zero_cnt(j, _):
        cnt_v[pl.ds(j * 16, 16)] = jnp.zeros((16,), jnp.int32)
        return 0
    lax.fori_loop(0, CPW // 16, zero_cnt, 0)

    @pl.when(t > 0)
    def _general():
        pltpu.sync_copy(ent_hbm, ent_v)
        pltpu.sync_copy(yh_hbm, yh_v)
        lanes = lax.iota(jnp.int32, 16)
        inf = jnp.float32(jnp.inf)

        def per_class(kloc, _):
            k = base_k + kloc

            for j in range(D // 16):
                accrow_v[pl.ds(j * 16, 16)] = zeros16

            def w_cond(st):
                return st[0]

            def w_body(st):
                _, e_last, i_last, csel = st

                def scan_chunk(ci, carry):
                    emin, imin = carry
                    e = ent_v[pl.ds(ci * 16, 16)]
                    y = yh_v[pl.ds(ci * 16, 16)]
                    idx = ci * 16 + lanes
                    q = ((e <= ENT_THRESHOLD) & (y == k)
                         & ((e > e_last) | ((e == e_last) & (idx > i_last))))
                    ec = jnp.where(q, e, inf)
                    ic = jnp.where(q, idx, BIGI)
                    better = (ec < emin) | ((ec == emin) & (ic < imin))
                    return (jnp.where(better, ec, emin),
                            jnp.where(better, ic, imin))

                emin, imin = lax.fori_loop(
                    0, B // 16, scan_chunk,
                    (jnp.full((16,), inf, jnp.float32),
                     jnp.full((16,), BIGI, jnp.int32)))
                e_s = jnp.min(emin)
                i_s = jnp.min(jnp.where(emin == e_s, imin, BIGI))
                found = e_s < inf

                @pl.when(found)
                def _():
                    pltpu.sync_copy(zn_hbm.at[pl.ds(i_s * D, D)], row_v)
                    for j in range(D // 16):
                        accrow_v[pl.ds(j * 16, 16)] = (
                            accrow_v[pl.ds(j * 16, 16)]
                            + row_v[pl.ds(j * 16, 16)])

                csel2 = csel + jnp.where(found, 1, 0)
                cont = found & (csel2 < M - 1)
                return (cont,
                        jnp.where(found, e_s, e_last),
                        jnp.where(found, i_s, i_last),
                        csel2)

            st = lax.while_loop(
                w_cond, w_body,
                (jnp.array(True), jnp.float32(-jnp.inf),
                 jnp.int32(-1), jnp.int32(0)))
            cnt_v[kloc] = st[3]
            for j in range(D // 16):
                acc_v[pl.ds(kloc * D + j * 16, 16)] = accrow_v[pl.ds(j * 16, 16)]
            return 0

        lax.fori_loop(0, CPW, per_class, 0)

    pltpu.sync_copy(acc_v, acc_hbm.at[pl.ds(wid * CPW * D, CPW * D)])
    pltpu.sync_copy(cnt_v, cnt_hbm.at[pl.ds(wid * CPW, CPW)])


_sc = pl.kernel(
    _sc_body,
    out_type=[
        jax.ShapeDtypeStruct((KP * D,), jnp.float32),
        jax.ShapeDtypeStruct((KP,), jnp.int32),
    ],
    mesh=plsc.VectorSubcoreMesh(core_axis_name="c", subcore_axis_name="s"),
    scratch_types=[
        pltpu.VMEM((16,), jnp.int32),
        pltpu.VMEM((B,), jnp.float32),
        pltpu.VMEM((B,), jnp.int32),
        pltpu.VMEM((D,), jnp.float32),
        pltpu.VMEM((D,), jnp.float32),
        pltpu.VMEM((CPW * D,), jnp.float32),
        pltpu.VMEM((CPW,), jnp.int32),
    ],
)


def _k2_body(zn_ref, W_ref, acc_ref, cnt_ref, out_ref, Cn_ref):
    i = pl.program_id(0)

    @pl.when(i == 0)
    def _():
        Wm = W_ref[...]
        wn = jnp.sqrt(jnp.sum(Wm * Wm, axis=1, keepdims=True))
        Wn = Wm / jnp.maximum(wn, 1e-12)
        Cm = (Wn + acc_ref[...]) / (1.0 + cnt_ref[...].astype(jnp.float32))
        cn = jnp.sqrt(jnp.sum(Cm * Cm, axis=1, keepdims=True))
        Cn_ref[...] = Cm / jnp.maximum(cn, 1e-12)

    out_ref[...] = lax.dot_general(zn_ref[...], Cn_ref[...],
                                   (((1,), (1,)), ((), ())),
                                   preferred_element_type=jnp.float32)


_k2 = pl.pallas_call(
    _k2_body,
    grid=(NB,),
    in_specs=[
        pl.BlockSpec((BLK, D), lambda i: (i, 0)),
        pl.BlockSpec((K, D), lambda i: (0, 0)),
        pl.BlockSpec((K, D), lambda i: (0, 0)),
        pl.BlockSpec((K, 1), lambda i: (0, 0)),
    ],
    out_specs=pl.BlockSpec((BLK, K), lambda i: (i, 0)),
    out_shape=jax.ShapeDtypeStruct((B, K), jnp.float32),
    scratch_shapes=[pltpu.VMEM((K, D), jnp.float32)],
)


def kernel(z, W, b):
    ent2, yh2, zn, tot = _k1(z, W, b.reshape(1, K))
    tot16 = jnp.broadcast_to(tot.reshape(1), (16,))
    accf, cntp = _sc(tot16, ent2.reshape(B), yh2.reshape(B), zn.reshape(B * D))
    acc = accf.reshape(KP, D)[:K]
    cnt = cntp[:K].reshape(K, 1)
    return _k2(zn, W, acc, cnt)


# speedup vs baseline: 13.5060x; 13.5060x over previous
import jax
import jax.numpy as jnp
from jax import lax
from jax.experimental import pallas as pl
from jax.experimental.pallas import tpu as pltpu
from jax.experimental.pallas import tpu_sc as plsc

B, D, K, M = 16384, 128, 1000, 30
ENT_THRESHOLD = 0.6
KP = 1024
BLK = 512
NB = B // BLK

NC, NS = 2, 16
NW = NC * NS
CPW = KP // NW
BIGI = 2**30


def _k1_body(z_ref, W_ref, b_ref, ent_ref, yh_ref, zn_ref, tot_ref):
    i = pl.program_id(0)
    z = z_ref[...]
    Wm = W_ref[...]
    logits = lax.dot_general(z, Wm, (((1,), (1,)), ((), ())),
                             preferred_element_type=jnp.float32)
    logits = logits + b_ref[...]
    m = jnp.max(logits, axis=1, keepdims=True)
    ex = jnp.exp(logits - m)
    S = jnp.sum(ex, axis=1, keepdims=True)
    ent = jnp.log(S) - jnp.sum(ex * (logits - m), axis=1, keepdims=True) / S
    cols = lax.broadcasted_iota(jnp.int32, logits.shape, 1)
    yh = jnp.min(jnp.where(logits == m, cols, K), axis=1, keepdims=True)
    nrm = jnp.sqrt(jnp.sum(z * z, axis=1, keepdims=True))
    zn = z / jnp.maximum(nrm, 1e-12)

    ent_ref[...] = ent
    yh_ref[...] = yh
    zn_ref[...] = zn
    kc = jnp.sum((ent <= ENT_THRESHOLD).astype(jnp.int32))

    @pl.when(i == 0)
    def _():
        tot_ref[0, 0] = kc

    @pl.when(i > 0)
    def _():
        tot_ref[0, 0] += kc


_k1 = pl.pallas_call(
    _k1_body,
    grid=(NB,),
    in_specs=[
        pl.BlockSpec((BLK, D), lambda i: (i, 0)),
        pl.BlockSpec((K, D), lambda i: (0, 0)),
        pl.BlockSpec((1, K), lambda i: (0, 0)),
    ],
    out_specs=[
        pl.BlockSpec((BLK, 1), lambda i: (i, 0)),
        pl.BlockSpec((BLK, 1), lambda i: (i, 0)),
        pl.BlockSpec((BLK, D), lambda i: (i, 0)),
        pl.BlockSpec(memory_space=pltpu.SMEM),
    ],
    out_shape=[
        jax.ShapeDtypeStruct((B, 1), jnp.float32),
        jax.ShapeDtypeStruct((B, 1), jnp.int32),
        jax.ShapeDtypeStruct((B, D), jnp.float32),
        jax.ShapeDtypeStruct((1, 1), jnp.int32),
    ],
)


def _sc_body(tot_hbm, ent_hbm, yh_hbm, zn_hbm, acc_hbm, cnt_hbm,
             tot_v, ent_v, yh_v, row_v, accrow_v, acc_v, cnt_v):
    c = lax.axis_index("c")
    s = lax.axis_index("s")
    wid = s * NC + c
    base_k = wid * CPW

    pltpu.sync_copy(tot_hbm, tot_v)
    t = tot_v[...][0]

    zeros16 = jnp.zeros((16,), jnp.float32)

    def zero_acc(j, _):
        acc_v[pl.ds(j * 16, 16)] = zeros16
        return 0
    lax.fori_loop(0, CPW * D // 16, zero_acc, 0)

    def zero_cnt(j, _):
        cnt_v[pl.ds(j * 16, 16)] = jnp.zeros((16,), jnp.int32)
        return 0
    lax.fori_loop(0, CPW // 16, zero_cnt, 0)

    @pl.when(t > 0)
    def _general():
        pltpu.sync_copy(ent_hbm, ent_v)
        pltpu.sync_copy(yh_hbm, yh_v)
        lanes = lax.iota(jnp.int32, 16)
        inf = jnp.float32(jnp.inf)

        def per_class(kloc, cnt_chunks):
            k = base_k + kloc

            for j in range(D // 16):
                accrow_v[pl.ds(j * 16, 16)] = zeros16

            def round_body(r, st):
                e_last, i_last, csel = st

                def scan_chunk(ci, carry):
                    emin, imin = carry
                    e = ent_v[pl.ds(ci * 16, 16)]
                    y = yh_v[pl.ds(ci * 16, 16)]
                    idx = ci * 16 + lanes
                    q = ((e <= ENT_THRESHOLD) & (y == k)
                         & ((e > e_last) | ((e == e_last) & (idx > i_last))))
                    ec = jnp.where(q, e, inf)
                    ic = jnp.where(q, idx, BIGI)
                    better = (ec < emin) | ((ec == emin) & (ic < imin))
                    return (jnp.where(better, ec, emin),
                            jnp.where(better, ic, imin))

                emin, imin = lax.fori_loop(
                    0, B // 16, scan_chunk,
                    (jnp.full((16,), inf, jnp.float32),
                     jnp.full((16,), BIGI, jnp.int32)))
                e_s, i_s = inf, jnp.int32(BIGI)
                for j in range(16):
                    e, ii = emin[j], imin[j]
                    better = (e < e_s) | ((e == e_s) & (ii < i_s))
                    e_s = jnp.where(better, e, e_s)
                    i_s = jnp.where(better, ii, i_s)
                found = e_s < inf

                i_safe = jnp.where(found, i_s, 0)
                pltpu.sync_copy(zn_hbm.at[pl.ds(i_safe * D, D)], row_v)
                fmul = jnp.where(found, jnp.float32(1), jnp.float32(0))
                for j in range(D // 16):
                    accrow_v[pl.ds(j * 16, 16)] = (
                        accrow_v[pl.ds(j * 16, 16)]
                        + row_v[pl.ds(j * 16, 16)] * fmul)
                return (jnp.where(found, e_s, e_last),
                        jnp.where(found, i_s, i_last),
                        csel + jnp.where(found, 1, 0))

            _, _, csel = lax.fori_loop(
                0, M - 1, round_body,
                (jnp.float32(-jnp.inf), jnp.int32(-1), jnp.int32(0)))

            for j in range(D // 16):
                acc_v[pl.ds(kloc * D + j * 16, 16)] = accrow_v[pl.ds(j * 16, 16)]

            c0, c1 = cnt_chunks
            csplat = jnp.full((16,), csel, jnp.int32)
            tgt0 = jnp.where(kloc < 16, kloc, -1)
            tgt1 = jnp.where(kloc >= 16, kloc - 16, -1)
            c0 = jnp.where(lanes == tgt0, csplat, c0)
            c1 = jnp.where(lanes == tgt1, csplat, c1)
            return (c0, c1)

        c0, c1 = lax.fori_loop(
            0, CPW, per_class,
            (jnp.zeros((16,), jnp.int32), jnp.zeros((16,), jnp.int32)))
        cnt_v[pl.ds(0, 16)] = c0
        cnt_v[pl.ds(16, 16)] = c1

    pltpu.sync_copy(acc_v, acc_hbm.at[pl.ds(wid * CPW * D, CPW * D)])
    pltpu.sync_copy(cnt_v, cnt_hbm.at[pl.ds(wid * CPW, CPW)])


_sc = pl.kernel(
    _sc_body,
    out_type=[
        jax.ShapeDtypeStruct((KP * D,), jnp.float32),
        jax.ShapeDtypeStruct((KP,), jnp.int32),
    ],
    mesh=plsc.VectorSubcoreMesh(core_axis_name="c", subcore_axis_name="s",
                                num_cores=NC, num_subcores=NS),
    scratch_types=[
        pltpu.VMEM((16,), jnp.int32),
        pltpu.VMEM((B,), jnp.float32),
        pltpu.VMEM((B,), jnp.int32),
        pltpu.VMEM((D,), jnp.float32),
        pltpu.VMEM((D,), jnp.float32),
        pltpu.VMEM((CPW * D,), jnp.float32),
        pltpu.VMEM((CPW,), jnp.int32),
    ],
)


def _k2_body(zn_ref, W_ref, acc_ref, cnt_ref, out_ref, Cn_ref):
    i = pl.program_id(0)

    @pl.when(i == 0)
    def _():
        Wm = W_ref[...]
        wn = jnp.sqrt(jnp.sum(Wm * Wm, axis=1, keepdims=True))
        Wn = Wm / jnp.maximum(wn, 1e-12)
        Cm = (Wn + acc_ref[...]) / (1.0 + cnt_ref[...].astype(jnp.float32))
        cn = jnp.sqrt(jnp.sum(Cm * Cm, axis=1, keepdims=True))
        Cn_ref[...] = Cm / jnp.maximum(cn, 1e-12)

    out_ref[...] = lax.dot_general(zn_ref[...], Cn_ref[...],
                                   (((1,), (1,)), ((), ())),
                                   preferred_element_type=jnp.float32)


_k2 = pl.pallas_call(
    _k2_body,
    grid=(NB,),
    in_specs=[
        pl.BlockSpec((BLK, D), lambda i: (i, 0)),
        pl.BlockSpec((K, D), lambda i: (0, 0)),
        pl.BlockSpec((K, D), lambda i: (0, 0)),
        pl.BlockSpec((K, 1), lambda i: (0, 0)),
    ],
    out_specs=pl.BlockSpec((BLK, K), lambda i: (i, 0)),
    out_shape=jax.ShapeDtypeStruct((B, K), jnp.float32),
    scratch_shapes=[pltpu.VMEM((K, D), jnp.float32)],
)


def kernel(z, W, b):
    ent2, yh2, zn, tot = _k1(z, W, b.reshape(1, K))
    tot16 = jnp.broadcast_to(tot.reshape(1), (16,))
    accf, cntp = _sc(tot16, ent2.reshape(B), yh2.reshape(B), zn.reshape(B * D))
    acc = accf.reshape(KP, D)[:K]
    cnt = cntp[:K].reshape(K, 1)
    return _k2(zn, W, acc, cnt)


# speedup vs baseline: 14.8701x; 1.1010x over previous
import jax
import jax.numpy as jnp
from jax import lax
from jax.experimental import pallas as pl
from jax.experimental.pallas import tpu as pltpu
from jax.experimental.pallas import tpu_sc as plsc

B, D, K, M = 16384, 128, 1000, 30
ENT_THRESHOLD = 0.6
KP = 1024
BLK = 1024
NB = B // BLK

NC, NS = 2, 16
NW = NC * NS
CPW = KP // NW
BIGI = 2**30


def _k1_body(z_ref, W_ref, b_ref, ent_ref, yh_ref, zn_ref, tot_ref):
    i = pl.program_id(0)
    z = z_ref[...]
    Wm = W_ref[...]
    logits = lax.dot_general(z, Wm, (((1,), (1,)), ((), ())),
                             preferred_element_type=jnp.float32)
    logits = logits + b_ref[...]
    m = jnp.max(logits, axis=1, keepdims=True)
    ex = jnp.exp(logits - m)
    S = jnp.sum(ex, axis=1, keepdims=True)
    ent = jnp.log(S) - jnp.sum(ex * (logits - m), axis=1, keepdims=True) / S
    cols = lax.broadcasted_iota(jnp.int32, logits.shape, 1)
    yh = jnp.min(jnp.where(logits == m, cols, K), axis=1, keepdims=True)
    nrm = jnp.sqrt(jnp.sum(z * z, axis=1, keepdims=True))
    zn = z / jnp.maximum(nrm, 1e-12)

    ent_ref[...] = ent
    yh_ref[...] = yh
    zn_ref[...] = zn
    kc = jnp.sum((ent <= ENT_THRESHOLD).astype(jnp.int32))

    @pl.when(i == 0)
    def _():
        tot_ref[0, 0] = kc

    @pl.when(i > 0)
    def _():
        tot_ref[0, 0] += kc


_k1 = pl.pallas_call(
    _k1_body,
    grid=(NB,),
    in_specs=[
        pl.BlockSpec((BLK, D), lambda i: (i, 0)),
        pl.BlockSpec((K, D), lambda i: (0, 0)),
        pl.BlockSpec((1, K), lambda i: (0, 0)),
    ],
    out_specs=[
        pl.BlockSpec((BLK, 1), lambda i: (i, 0)),
        pl.BlockSpec((BLK, 1), lambda i: (i, 0)),
        pl.BlockSpec((BLK, D), lambda i: (i, 0)),
        pl.BlockSpec(memory_space=pltpu.SMEM),
    ],
    out_shape=[
        jax.ShapeDtypeStruct((B, 1), jnp.float32),
        jax.ShapeDtypeStruct((B, 1), jnp.int32),
        jax.ShapeDtypeStruct((B, D), jnp.float32),
        jax.ShapeDtypeStruct((1, 1), jnp.int32),
    ],
)


def _sc_body(tot_hbm, ent_hbm, yh_hbm, zn_hbm, acc_hbm, cnt_hbm,
             tot_v, ent_v, yh_v, row_v, accrow_v, acc_v, cnt_v):
    c = lax.axis_index("c")
    s = lax.axis_index("s")
    wid = s * NC + c
    base_k = wid * CPW

    pltpu.sync_copy(tot_hbm, tot_v)
    t = tot_v[...][0]

    zeros16 = jnp.zeros((16,), jnp.float32)

    def zero_acc(j, _):
        acc_v[pl.ds(j * 16, 16)] = zeros16
        return 0
    lax.fori_loop(0, CPW * D // 16, zero_acc, 0)

    def zero_cnt(j, _):
        cnt_v[pl.ds(j * 16, 16)] = jnp.zeros((16,), jnp.int32)
        return 0
    lax.fori_loop(0, CPW // 16, zero_cnt, 0)

    @pl.when(t > 0)
    def _general():
        pltpu.sync_copy(ent_hbm, ent_v)
        pltpu.sync_copy(yh_hbm, yh_v)
        lanes = lax.iota(jnp.int32, 16)
        inf = jnp.float32(jnp.inf)

        def per_class(kloc, cnt_chunks):
            k = base_k + kloc

            for j in range(D // 16):
                accrow_v[pl.ds(j * 16, 16)] = zeros16

            def round_body(r, st):
                e_last, i_last, csel = st

                def scan_chunk(ci, carry):
                    emin, imin = carry
                    e = ent_v[pl.ds(ci * 16, 16)]
                    y = yh_v[pl.ds(ci * 16, 16)]
                    idx = ci * 16 + lanes
                    q = ((e <= ENT_THRESHOLD) & (y == k)
                         & ((e > e_last) | ((e == e_last) & (idx > i_last))))
                    ec = jnp.where(q, e, inf)
                    ic = jnp.where(q, idx, BIGI)
                    better = (ec < emin) | ((ec == emin) & (ic < imin))
                    return (jnp.where(better, ec, emin),
                            jnp.where(better, ic, imin))

                emin, imin = lax.fori_loop(
                    0, B // 16, scan_chunk,
                    (jnp.full((16,), inf, jnp.float32),
                     jnp.full((16,), BIGI, jnp.int32)))
                e_s, i_s = inf, jnp.int32(BIGI)
                for j in range(16):
                    e, ii = emin[j], imin[j]
                    better = (e < e_s) | ((e == e_s) & (ii < i_s))
                    e_s = jnp.where(better, e, e_s)
                    i_s = jnp.where(better, ii, i_s)
                found = e_s < inf

                i_safe = jnp.where(found, i_s, 0)
                pltpu.sync_copy(zn_hbm.at[pl.ds(i_safe * D, D)], row_v)
                fmul = jnp.where(found, jnp.float32(1), jnp.float32(0))
                for j in range(D // 16):
                    accrow_v[pl.ds(j * 16, 16)] = (
                        accrow_v[pl.ds(j * 16, 16)]
                        + row_v[pl.ds(j * 16, 16)] * fmul)
                return (jnp.where(found, e_s, e_last),
                        jnp.where(found, i_s, i_last),
                        csel + jnp.where(found, 1, 0))

            _, _, csel = lax.fori_loop(
                0, M - 1, round_body,
                (jnp.float32(-jnp.inf), jnp.int32(-1), jnp.int32(0)))

            for j in range(D // 16):
                acc_v[pl.ds(kloc * D + j * 16, 16)] = accrow_v[pl.ds(j * 16, 16)]

            c0, c1 = cnt_chunks
            csplat = jnp.full((16,), csel, jnp.int32)
            tgt0 = jnp.where(kloc < 16, kloc, -1)
            tgt1 = jnp.where(kloc >= 16, kloc - 16, -1)
            c0 = jnp.where(lanes == tgt0, csplat, c0)
            c1 = jnp.where(lanes == tgt1, csplat, c1)
            return (c0, c1)

        c0, c1 = lax.fori_loop(
            0, CPW, per_class,
            (jnp.zeros((16,), jnp.int32), jnp.zeros((16,), jnp.int32)))
        cnt_v[pl.ds(0, 16)] = c0
        cnt_v[pl.ds(16, 16)] = c1

    pltpu.sync_copy(acc_v, acc_hbm.at[pl.ds(wid * CPW * D, CPW * D)])
    pltpu.sync_copy(cnt_v, cnt_hbm.at[pl.ds(wid * CPW, CPW)])


_sc = pl.kernel(
    _sc_body,
    out_type=[
        jax.ShapeDtypeStruct((KP * D,), jnp.float32),
        jax.ShapeDtypeStruct((KP,), jnp.int32),
    ],
    mesh=plsc.VectorSubcoreMesh(core_axis_name="c", subcore_axis_name="s",
                                num_cores=NC, num_subcores=NS),
    scratch_types=[
        pltpu.VMEM((16,), jnp.int32),
        pltpu.VMEM((B,), jnp.float32),
        pltpu.VMEM((B,), jnp.int32),
        pltpu.VMEM((D,), jnp.float32),
        pltpu.VMEM((D,), jnp.float32),
        pltpu.VMEM((CPW * D,), jnp.float32),
        pltpu.VMEM((CPW,), jnp.int32),
    ],
)


def _k2_body(zn_ref, W_ref, acc_ref, cnt_ref, out_ref, Cn_ref):
    i = pl.program_id(0)

    @pl.when(i == 0)
    def _():
        Wm = W_ref[...]
        wn = jnp.sqrt(jnp.sum(Wm * Wm, axis=1, keepdims=True))
        Wn = Wm / jnp.maximum(wn, 1e-12)
        Cm = (Wn + acc_ref[...]) / (1.0 + cnt_ref[...].astype(jnp.float32))
        cn = jnp.sqrt(jnp.sum(Cm * Cm, axis=1, keepdims=True))
        Cn_ref[...] = Cm / jnp.maximum(cn, 1e-12)

    out_ref[...] = lax.dot_general(zn_ref[...], Cn_ref[...],
                                   (((1,), (1,)), ((), ())),
                                   preferred_element_type=jnp.float32)


_k2 = pl.pallas_call(
    _k2_body,
    grid=(NB,),
    in_specs=[
        pl.BlockSpec((BLK, D), lambda i: (i, 0)),
        pl.BlockSpec((K, D), lambda i: (0, 0)),
        pl.BlockSpec((K, D), lambda i: (0, 0)),
        pl.BlockSpec((K, 1), lambda i: (0, 0)),
    ],
    out_specs=pl.BlockSpec((BLK, K), lambda i: (i, 0)),
    out_shape=jax.ShapeDtypeStruct((B, K), jnp.float32),
    scratch_shapes=[pltpu.VMEM((K, D), jnp.float32)],
)


def kernel(z, W, b):
    ent2, yh2, zn, tot = _k1(z, W, b.reshape(1, K))
    tot16 = jnp.broadcast_to(tot.reshape(1), (16,))
    accf, cntp = _sc(tot16, ent2.reshape(B), yh2.reshape(B), zn.reshape(B * D))
    return _k2(zn, W, accf.reshape(KP, D), cntp.reshape(KP, 1))


# speedup vs baseline: 15.0995x; 1.0154x over previous
import jax
import jax.numpy as jnp
from jax import lax
from jax.experimental import pallas as pl
from jax.experimental.pallas import tpu as pltpu
from jax.experimental.pallas import tpu_sc as plsc

B, D, K, M = 16384, 128, 1000, 30
ENT_THRESHOLD = 0.6
KP = 1024
BLK = 2048
NB = B // BLK

NC, NS = 2, 16
NW = NC * NS
CPW = KP // NW
BIGI = 2**30


def _k1_body(z_ref, W_ref, b_ref, ent_ref, yh_ref, zn_ref, tot_ref):
    i = pl.program_id(0)
    z = z_ref[...]
    Wm = W_ref[...]
    logits = lax.dot_general(z, Wm, (((1,), (1,)), ((), ())),
                             preferred_element_type=jnp.float32)
    logits = logits + b_ref[...]
    m = jnp.max(logits, axis=1, keepdims=True)
    ex = jnp.exp(logits - m)
    S = jnp.sum(ex, axis=1, keepdims=True)
    ent = jnp.log(S) - jnp.sum(ex * (logits - m), axis=1, keepdims=True) / S
    cols = lax.broadcasted_iota(jnp.int32, logits.shape, 1)
    yh = jnp.min(jnp.where(logits == m, cols, K), axis=1, keepdims=True)
    nrm = jnp.sqrt(jnp.sum(z * z, axis=1, keepdims=True))
    zn = z / jnp.maximum(nrm, 1e-12)

    ent_ref[...] = ent
    yh_ref[...] = yh
    zn_ref[...] = zn
    kc = jnp.sum((ent <= ENT_THRESHOLD).astype(jnp.int32))

    @pl.when(i == 0)
    def _():
        tot_ref[0, 0] = kc

    @pl.when(i > 0)
    def _():
        tot_ref[0, 0] += kc


_k1 = pl.pallas_call(
    _k1_body,
    grid=(NB,),
    in_specs=[
        pl.BlockSpec((BLK, D), lambda i: (i, 0)),
        pl.BlockSpec((K, D), lambda i: (0, 0)),
        pl.BlockSpec((1, K), lambda i: (0, 0)),
    ],
    out_specs=[
        pl.BlockSpec((BLK, 1), lambda i: (i, 0)),
        pl.BlockSpec((BLK, 1), lambda i: (i, 0)),
        pl.BlockSpec((BLK, D), lambda i: (i, 0)),
        pl.BlockSpec(memory_space=pltpu.SMEM),
    ],
    out_shape=[
        jax.ShapeDtypeStruct((B, 1), jnp.float32),
        jax.ShapeDtypeStruct((B, 1), jnp.int32),
        jax.ShapeDtypeStruct((B, D), jnp.float32),
        jax.ShapeDtypeStruct((1, 1), jnp.int32),
    ],
)


def _sc_body(tot_hbm, ent_hbm, yh_hbm, zn_hbm, acc_hbm, cnt_hbm,
             tot_v, ent_v, yh_v, row_v, accrow_v, acc_v, cnt_v):
    c = lax.axis_index("c")
    s = lax.axis_index("s")
    wid = s * NC + c
    base_k = wid * CPW

    pltpu.sync_copy(tot_hbm, tot_v)
    t = tot_v[...][0]

    zeros16 = jnp.zeros((16,), jnp.float32)

    def zero_acc(j, _):
        acc_v[pl.ds(j * 16, 16)] = zeros16
        return 0
    lax.fori_loop(0, CPW * D // 16, zero_acc, 0)

    def zero_cnt(j, _):
        cnt_v[pl.ds(j * 16, 16)] = jnp.zeros((16,), jnp.int32)
        return 0
    lax.fori_loop(0, CPW // 16, zero_cnt, 0)

    @pl.when(t > 0)
    def _general():
        pltpu.sync_copy(ent_hbm, ent_v)
        pltpu.sync_copy(yh_hbm, yh_v)
        lanes = lax.iota(jnp.int32, 16)
        inf = jnp.float32(jnp.inf)

        def per_class(kloc, cnt_chunks):
            k = base_k + kloc

            for j in range(D // 16):
                accrow_v[pl.ds(j * 16, 16)] = zeros16

            def round_body(r, st):
                e_last, i_last, csel = st

                def scan_chunk(ci, carry):
                    emin, imin = carry
                    e = ent_v[pl.ds(ci * 16, 16)]
                    y = yh_v[pl.ds(ci * 16, 16)]
                    idx = ci * 16 + lanes
                    q = ((e <= ENT_THRESHOLD) & (y == k)
                         & ((e > e_last) | ((e == e_last) & (idx > i_last))))
                    ec = jnp.where(q, e, inf)
                    ic = jnp.where(q, idx, BIGI)
                    better = (ec < emin) | ((ec == emin) & (ic < imin))
                    return (jnp.where(better, ec, emin),
                            jnp.where(better, ic, imin))

                emin, imin = lax.fori_loop(
                    0, B // 16, scan_chunk,
                    (jnp.full((16,), inf, jnp.float32),
                     jnp.full((16,), BIGI, jnp.int32)))
                e_s, i_s = inf, jnp.int32(BIGI)
                for j in range(16):
                    e, ii = emin[j], imin[j]
                    better = (e < e_s) | ((e == e_s) & (ii < i_s))
                    e_s = jnp.where(better, e, e_s)
                    i_s = jnp.where(better, ii, i_s)
                found = e_s < inf

                i_safe = jnp.where(found, i_s, 0)
                pltpu.sync_copy(zn_hbm.at[pl.ds(i_safe * D, D)], row_v)
                fmul = jnp.where(found, jnp.float32(1), jnp.float32(0))
                for j in range(D // 16):
                    accrow_v[pl.ds(j * 16, 16)] = (
                        accrow_v[pl.ds(j * 16, 16)]
                        + row_v[pl.ds(j * 16, 16)] * fmul)
                return (jnp.where(found, e_s, e_last),
                        jnp.where(found, i_s, i_last),
                        csel + jnp.where(found, 1, 0))

            _, _, csel = lax.fori_loop(
                0, M - 1, round_body,
                (jnp.float32(-jnp.inf), jnp.int32(-1), jnp.int32(0)))

            for j in range(D // 16):
                acc_v[pl.ds(kloc * D + j * 16, 16)] = accrow_v[pl.ds(j * 16, 16)]

            c0, c1 = cnt_chunks
            csplat = jnp.full((16,), csel, jnp.int32)
            tgt0 = jnp.where(kloc < 16, kloc, -1)
            tgt1 = jnp.where(kloc >= 16, kloc - 16, -1)
            c0 = jnp.where(lanes == tgt0, csplat, c0)
            c1 = jnp.where(lanes == tgt1, csplat, c1)
            return (c0, c1)

        c0, c1 = lax.fori_loop(
            0, CPW, per_class,
            (jnp.zeros((16,), jnp.int32), jnp.zeros((16,), jnp.int32)))
        cnt_v[pl.ds(0, 16)] = c0
        cnt_v[pl.ds(16, 16)] = c1

    pltpu.sync_copy(acc_v, acc_hbm.at[pl.ds(wid * CPW * D, CPW * D)])
    pltpu.sync_copy(cnt_v, cnt_hbm.at[pl.ds(wid * CPW, CPW)])


_sc = pl.kernel(
    _sc_body,
    out_type=[
        jax.ShapeDtypeStruct((KP * D,), jnp.float32),
        jax.ShapeDtypeStruct((KP,), jnp.int32),
    ],
    mesh=plsc.VectorSubcoreMesh(core_axis_name="c", subcore_axis_name="s",
                                num_cores=NC, num_subcores=NS),
    scratch_types=[
        pltpu.VMEM((16,), jnp.int32),
        pltpu.VMEM((B,), jnp.float32),
        pltpu.VMEM((B,), jnp.int32),
        pltpu.VMEM((D,), jnp.float32),
        pltpu.VMEM((D,), jnp.float32),
        pltpu.VMEM((CPW * D,), jnp.float32),
        pltpu.VMEM((CPW,), jnp.int32),
    ],
)


def _k2_body(zn_ref, W_ref, acc_ref, cnt_ref, out_ref, Cn_ref):
    i = pl.program_id(0)

    @pl.when(i == 0)
    def _():
        Wm = W_ref[...]
        wn = jnp.sqrt(jnp.sum(Wm * Wm, axis=1, keepdims=True))
        Wn = Wm / jnp.maximum(wn, 1e-12)
        Cm = (Wn + acc_ref[...]) / (1.0 + cnt_ref[...].astype(jnp.float32))
        cn = jnp.sqrt(jnp.sum(Cm * Cm, axis=1, keepdims=True))
        Cn_ref[...] = Cm / jnp.maximum(cn, 1e-12)

    out_ref[...] = lax.dot_general(zn_ref[...], Cn_ref[...],
                                   (((1,), (1,)), ((), ())),
                                   preferred_element_type=jnp.float32)


_k2 = pl.pallas_call(
    _k2_body,
    grid=(NB,),
    in_specs=[
        pl.BlockSpec((BLK, D), lambda i: (i, 0)),
        pl.BlockSpec((K, D), lambda i: (0, 0)),
        pl.BlockSpec((K, D), lambda i: (0, 0)),
        pl.BlockSpec((K, 1), lambda i: (0, 0)),
    ],
    out_specs=pl.BlockSpec((BLK, K), lambda i: (i, 0)),
    out_shape=jax.ShapeDtypeStruct((B, K), jnp.float32),
    scratch_shapes=[pltpu.VMEM((K, D), jnp.float32)],
)


def kernel(z, W, b):
    ent2, yh2, zn, tot = _k1(z, W, b.reshape(1, K))
    tot16 = jnp.broadcast_to(tot.reshape(1), (16,))
    accf, cntp = _sc(tot16, ent2.reshape(B), yh2.reshape(B), zn.reshape(B * D))
    return _k2(zn, W, accf.reshape(KP, D), cntp.reshape(KP, 1))


# speedup vs baseline: 15.2055x; 1.0070x over previous
import jax
import jax.numpy as jnp
from jax import lax
from jax.experimental import pallas as pl
from jax.experimental.pallas import tpu as pltpu
from jax.experimental.pallas import tpu_sc as plsc

B, D, K, M = 16384, 128, 1000, 30
ENT_THRESHOLD = 0.6
KP = 1024
BLK = 2048
NB = B // BLK

NC, NS = 2, 16
NW = NC * NS
CPW = KP // NW
BIGI = 2**30


def _k1_body(z_ref, W_ref, b_ref, ent_ref, yh_ref, zn_ref, tot_ref):
    i = pl.program_id(0)
    z = z_ref[...]
    Wm = W_ref[...]
    logits = lax.dot_general(z.astype(jnp.bfloat16), Wm.astype(jnp.bfloat16),
                             (((1,), (1,)), ((), ())),
                             preferred_element_type=jnp.float32)
    logits = logits + b_ref[...]
    m = jnp.max(logits, axis=1, keepdims=True)
    ex = jnp.exp(logits - m)
    S = jnp.sum(ex, axis=1, keepdims=True)
    ent = jnp.log(S) - jnp.sum(ex * (logits - m), axis=1, keepdims=True) / S
    cols = lax.broadcasted_iota(jnp.int32, logits.shape, 1)
    yh = jnp.min(jnp.where(logits == m, cols, K), axis=1, keepdims=True)
    nrm = jnp.sqrt(jnp.sum(z * z, axis=1, keepdims=True))
    zn = z / jnp.maximum(nrm, 1e-12)

    ent_ref[...] = ent
    yh_ref[...] = yh
    zn_ref[...] = zn
    kc = jnp.sum((ent <= ENT_THRESHOLD).astype(jnp.int32))

    @pl.when(i == 0)
    def _():
        tot_ref[0, 0] = kc

    @pl.when(i > 0)
    def _():
        tot_ref[0, 0] += kc


_k1 = pl.pallas_call(
    _k1_body,
    grid=(NB,),
    in_specs=[
        pl.BlockSpec((BLK, D), lambda i: (i, 0)),
        pl.BlockSpec((K, D), lambda i: (0, 0)),
        pl.BlockSpec((1, K), lambda i: (0, 0)),
    ],
    out_specs=[
        pl.BlockSpec((BLK, 1), lambda i: (i, 0)),
        pl.BlockSpec((BLK, 1), lambda i: (i, 0)),
        pl.BlockSpec((BLK, D), lambda i: (i, 0)),
        pl.BlockSpec(memory_space=pltpu.SMEM),
    ],
    out_shape=[
        jax.ShapeDtypeStruct((B, 1), jnp.float32),
        jax.ShapeDtypeStruct((B, 1), jnp.int32),
        jax.ShapeDtypeStruct((B, D), jnp.float32),
        jax.ShapeDtypeStruct((1, 1), jnp.int32),
    ],
)


def _sc_body(tot_hbm, ent_hbm, yh_hbm, zn_hbm, acc_hbm, cnt_hbm,
             tot_v, ent_v, yh_v, row_v, accrow_v, acc_v, cnt_v):
    c = lax.axis_index("c")
    s = lax.axis_index("s")
    wid = s * NC + c
    base_k = wid * CPW

    pltpu.sync_copy(tot_hbm, tot_v)
    t = tot_v[...][0]

    zeros16 = jnp.zeros((16,), jnp.float32)

    def zero_acc(j, _):
        acc_v[pl.ds(j * 16, 16)] = zeros16
        return 0
    lax.fori_loop(0, CPW * D // 16, zero_acc, 0)

    def zero_cnt(j, _):
        cnt_v[pl.ds(j * 16, 16)] = jnp.zeros((16,), jnp.int32)
        return 0
    lax.fori_loop(0, CPW // 16, zero_cnt, 0)

    @pl.when(t > 0)
    def _general():
        pltpu.sync_copy(ent_hbm, ent_v)
        pltpu.sync_copy(yh_hbm, yh_v)
        lanes = lax.iota(jnp.int32, 16)
        inf = jnp.float32(jnp.inf)

        def per_class(kloc, cnt_chunks):
            k = base_k + kloc

            for j in range(D // 16):
                accrow_v[pl.ds(j * 16, 16)] = zeros16

            def round_body(r, st):
                e_last, i_last, csel = st

                def scan_chunk(ci, carry):
                    emin, imin = carry
                    e = ent_v[pl.ds(ci * 16, 16)]
                    y = yh_v[pl.ds(ci * 16, 16)]
                    idx = ci * 16 + lanes
                    q = ((e <= ENT_THRESHOLD) & (y == k)
                         & ((e > e_last) | ((e == e_last) & (idx > i_last))))
                    ec = jnp.where(q, e, inf)
                    ic = jnp.where(q, idx, BIGI)
                    better = (ec < emin) | ((ec == emin) & (ic < imin))
                    return (jnp.where(better, ec, emin),
                            jnp.where(better, ic, imin))

                emin, imin = lax.fori_loop(
                    0, B // 16, scan_chunk,
                    (jnp.full((16,), inf, jnp.float32),
                     jnp.full((16,), BIGI, jnp.int32)))
                e_s, i_s = inf, jnp.int32(BIGI)
                for j in range(16):
                    e, ii = emin[j], imin[j]
                    better = (e < e_s) | ((e == e_s) & (ii < i_s))
                    e_s = jnp.where(better, e, e_s)
                    i_s = jnp.where(better, ii, i_s)
                found = e_s < inf

                i_safe = jnp.where(found, i_s, 0)
                pltpu.sync_copy(zn_hbm.at[pl.ds(i_safe * D, D)], row_v)
                fmul = jnp.where(found, jnp.float32(1), jnp.float32(0))
                for j in range(D // 16):
                    accrow_v[pl.ds(j * 16, 16)] = (
                        accrow_v[pl.ds(j * 16, 16)]
                        + row_v[pl.ds(j * 16, 16)] * fmul)
                return (jnp.where(found, e_s, e_last),
                        jnp.where(found, i_s, i_last),
                        csel + jnp.where(found, 1, 0))

            _, _, csel = lax.fori_loop(
                0, M - 1, round_body,
                (jnp.float32(-jnp.inf), jnp.int32(-1), jnp.int32(0)))

            for j in range(D // 16):
                acc_v[pl.ds(kloc * D + j * 16, 16)] = accrow_v[pl.ds(j * 16, 16)]

            c0, c1 = cnt_chunks
            csplat = jnp.full((16,), csel, jnp.int32)
            tgt0 = jnp.where(kloc < 16, kloc, -1)
            tgt1 = jnp.where(kloc >= 16, kloc - 16, -1)
            c0 = jnp.where(lanes == tgt0, csplat, c0)
            c1 = jnp.where(lanes == tgt1, csplat, c1)
            return (c0, c1)

        c0, c1 = lax.fori_loop(
            0, CPW, per_class,
            (jnp.zeros((16,), jnp.int32), jnp.zeros((16,), jnp.int32)))
        cnt_v[pl.ds(0, 16)] = c0
        cnt_v[pl.ds(16, 16)] = c1

    pltpu.sync_copy(acc_v, acc_hbm.at[pl.ds(wid * CPW * D, CPW * D)])
    pltpu.sync_copy(cnt_v, cnt_hbm.at[pl.ds(wid * CPW, CPW)])


_sc = pl.kernel(
    _sc_body,
    out_type=[
        jax.ShapeDtypeStruct((KP * D,), jnp.float32),
        jax.ShapeDtypeStruct((KP,), jnp.int32),
    ],
    mesh=plsc.VectorSubcoreMesh(core_axis_name="c", subcore_axis_name="s",
                                num_cores=NC, num_subcores=NS),
    scratch_types=[
        pltpu.VMEM((16,), jnp.int32),
        pltpu.VMEM((B,), jnp.float32),
        pltpu.VMEM((B,), jnp.int32),
        pltpu.VMEM((D,), jnp.float32),
        pltpu.VMEM((D,), jnp.float32),
        pltpu.VMEM((CPW * D,), jnp.float32),
        pltpu.VMEM((CPW,), jnp.int32),
    ],
)


def _k2_body(zn_ref, W_ref, acc_ref, cnt_ref, out_ref, Cn_ref):
    i = pl.program_id(0)

    @pl.when(i == 0)
    def _():
        Wm = W_ref[...]
        wn = jnp.sqrt(jnp.sum(Wm * Wm, axis=1, keepdims=True))
        Wn = Wm / jnp.maximum(wn, 1e-12)
        Cm = (Wn + acc_ref[...]) / (1.0 + cnt_ref[...].astype(jnp.float32))
        cn = jnp.sqrt(jnp.sum(Cm * Cm, axis=1, keepdims=True))
        Cn_ref[...] = Cm / jnp.maximum(cn, 1e-12)

    out_ref[...] = lax.dot_general(zn_ref[...].astype(jnp.bfloat16),
                                   Cn_ref[...].astype(jnp.bfloat16),
                                   (((1,), (1,)), ((), ())),
                                   preferred_element_type=jnp.float32)


_k2 = pl.pallas_call(
    _k2_body,
    grid=(NB,),
    in_specs=[
        pl.BlockSpec((BLK, D), lambda i: (i, 0)),
        pl.BlockSpec((K, D), lambda i: (0, 0)),
        pl.BlockSpec((K, D), lambda i: (0, 0)),
        pl.BlockSpec((K, 1), lambda i: (0, 0)),
    ],
    out_specs=pl.BlockSpec((BLK, K), lambda i: (i, 0)),
    out_shape=jax.ShapeDtypeStruct((B, K), jnp.float32),
    scratch_shapes=[pltpu.VMEM((K, D), jnp.float32)],
)


def kernel(z, W, b):
    ent2, yh2, zn, tot = _k1(z, W, b.reshape(1, K))
    tot16 = jnp.broadcast_to(tot.reshape(1), (16,))
    accf, cntp = _sc(tot16, ent2.reshape(B), yh2.reshape(B), zn.reshape(B * D))
    return _k2(zn, W, accf.reshape(KP, D), cntp.reshape(KP, 1))
